# baseline (speedup 1.0000x reference)
import functools

import jax
import jax.numpy as jnp
from jax.experimental import pallas as pl
from jax.experimental.pallas import tpu as pltpu

_OBS = 8
_ACT = 4
_HID = 64
_PACK = 16
_T16 = 512
_STEP = _PACK * _T16


def _ac_kernel(x_ref, w1_ref, b1_ref, w2_ref, b2_ref, wh_ref, bh_ref,
               out_ref):
    h = jnp.tanh(
        jnp.dot(x_ref[...], w1_ref[...], preferred_element_type=jnp.float32)
        + b1_ref[...]
    )
    h = jnp.tanh(
        jnp.dot(h, w2_ref[...], preferred_element_type=jnp.float32)
        + b2_ref[...]
    )
    out_ref[...] = (
        jnp.dot(h, wh_ref[...], preferred_element_type=jnp.float32)
        + bh_ref[...]
    )


@functools.partial(jax.jit, static_argnames=("rows",))
def _forward(xd, w1b, b1b, w2b, b2b, whb, bhb, *, rows):
    grid = (rows // _T16,)
    return pl.pallas_call(
        _ac_kernel,
        grid=grid,
        in_specs=[
            pl.BlockSpec((_T16, 128), lambda i: (i, 0)),
            pl.BlockSpec((128, 1024), lambda i: (0, 0)),
            pl.BlockSpec((1, 1024), lambda i: (0, 0)),
            pl.BlockSpec((1024, 1024), lambda i: (0, 0)),
            pl.BlockSpec((1, 1024), lambda i: (0, 0)),
            pl.BlockSpec((1024, 128), lambda i: (0, 0)),
            pl.BlockSpec((1, 128), lambda i: (0, 0)),
        ],
        out_specs=pl.BlockSpec((_T16, 128), lambda i: (i, 0)),
        out_shape=jax.ShapeDtypeStruct((rows, 128), jnp.float32),
        compiler_params=pltpu.CompilerParams(
            dimension_semantics=("parallel",),
        ),
    )(xd, w1b, b1b, w2b, b2b, whb, bhb)


def kernel(x, w1, b1, w2, b2, wh, bh):
    B = x.shape[0]
    Bp = -(-B // _STEP) * _STEP
    if Bp != B:
        x = jnp.pad(x, ((0, Bp - B), (0, 0)))
    xd = x.reshape(Bp // _PACK, _PACK * _OBS)

    eye = jnp.eye(_PACK, dtype=jnp.float32)
    w1b = jnp.kron(eye, w1[:, :_HID])
    b1b = jnp.tile(b1[:, :_HID], (1, _PACK))
    w2b = jnp.kron(eye, w2[:_HID, :_HID])
    b2b = jnp.tile(b2[:, :_HID], (1, _PACK))
    whb = jnp.kron(eye, wh[:_HID, :])
    bhb = jnp.tile(bh, (1, _PACK))

    rows = Bp // _PACK
    out = _forward(xd, w1b, b1b, w2b, b2b, whb, bhb, rows=rows)
    o8 = out.reshape(Bp, _OBS)
    logits = o8[:B, :_ACT]
    value = o8[:B, _ACT:_ACT + 1]
    return logits, value


# speedup vs baseline: 2.0171x; 2.0171x over previous
import functools

import jax
import jax.numpy as jnp
from jax.experimental import pallas as pl
from jax.experimental.pallas import tpu as pltpu

_OBS = 8
_ACT = 4
_HID = 64
_TILE = 4096


def _ac_kernel(x_ref, w1a_ref, w1b_ref, b1_ref, w2_ref, b2_ref,
               wla_ref, wlb_ref, bl_ref, wva_ref, wvb_ref, bv_ref,
               logits_ref, value_ref):
    xa = x_ref[0]
    xb = x_ref[1]
    h1 = jnp.tanh(
        jnp.dot(xa, w1a_ref[...], preferred_element_type=jnp.float32)
        + jnp.dot(xb, w1b_ref[...], preferred_element_type=jnp.float32)
        + b1_ref[...]
    )
    h2 = jnp.tanh(
        jnp.dot(h1, w2_ref[...], preferred_element_type=jnp.float32)
        + b2_ref[...]
    )
    logits_ref[0] = (
        jnp.dot(h2, wla_ref[...], preferred_element_type=jnp.float32)
        + bl_ref[...]
    )
    logits_ref[1] = (
        jnp.dot(h2, wlb_ref[...], preferred_element_type=jnp.float32)
        + bl_ref[...]
    )
    value_ref[0] = (
        jnp.dot(h2, wva_ref[...], preferred_element_type=jnp.float32)
        + bv_ref[...]
    )
    value_ref[1] = (
        jnp.dot(h2, wvb_ref[...], preferred_element_type=jnp.float32)
        + bv_ref[...]
    )


@functools.partial(jax.jit, static_argnames=("half",))
def _forward(x3, w1a, w1b, b1p, w2p, b2p, wla, wlb, bl, wva, wvb, bv, *, half):
    grid = (half // _TILE,)
    logits3, value3 = pl.pallas_call(
        _ac_kernel,
        grid=grid,
        in_specs=[
            pl.BlockSpec((2, _TILE, _OBS), lambda i: (0, i, 0)),
            pl.BlockSpec((_OBS, 128), lambda i: (0, 0)),
            pl.BlockSpec((_OBS, 128), lambda i: (0, 0)),
            pl.BlockSpec((1, 128), lambda i: (0, 0)),
            pl.BlockSpec((128, 128), lambda i: (0, 0)),
            pl.BlockSpec((1, 128), lambda i: (0, 0)),
            pl.BlockSpec((128, _ACT), lambda i: (0, 0)),
            pl.BlockSpec((128, _ACT), lambda i: (0, 0)),
            pl.BlockSpec((1, _ACT), lambda i: (0, 0)),
            pl.BlockSpec((128, 1), lambda i: (0, 0)),
            pl.BlockSpec((128, 1), lambda i: (0, 0)),
            pl.BlockSpec((1, 1), lambda i: (0, 0)),
        ],
        out_specs=[
            pl.BlockSpec((2, _TILE, _ACT), lambda i: (0, i, 0)),
            pl.BlockSpec((2, _TILE, 1), lambda i: (0, i, 0)),
        ],
        out_shape=[
            jax.ShapeDtypeStruct((2, half, _ACT), jnp.float32),
            jax.ShapeDtypeStruct((2, half, 1), jnp.float32),
        ],
        compiler_params=pltpu.CompilerParams(
            dimension_semantics=("parallel",),
        ),
    )(x3, w1a, w1b, b1p, w2p, b2p, wla, wlb, bl, wva, wvb, bv)
    return logits3, value3


def kernel(x, w1, b1, w2, b2, wh, bh):
    B = x.shape[0]
    half = -(-B // (2 * _TILE)) * _TILE
    if 2 * half != B:
        x = jnp.pad(x, ((0, 2 * half - B), (0, 0)))
    x3 = x.reshape(2, half, _OBS)

    w1c = w1[:, :_HID]
    w1a = jnp.zeros((_OBS, 128), jnp.float32).at[:, :_HID].set(w1c)
    w1b = jnp.zeros((_OBS, 128), jnp.float32).at[:, _HID:].set(w1c)
    b1c = b1[:, :_HID]
    b1p = jnp.concatenate([b1c, b1c], axis=1)
    w2c = w2[:_HID, :_HID]
    w2p = (jnp.zeros((128, 128), jnp.float32)
           .at[:_HID, :_HID].set(w2c)
           .at[_HID:, _HID:].set(w2c))
    b2c = b2[:, :_HID]
    b2p = jnp.concatenate([b2c, b2c], axis=1)
    wa = wh[:_HID, :_ACT]
    wla = jnp.zeros((128, _ACT), jnp.float32).at[:_HID].set(wa)
    wlb = jnp.zeros((128, _ACT), jnp.float32).at[_HID:].set(wa)
    bl = bh[:, :_ACT]
    wc = wh[:_HID, _ACT:_ACT + 1]
    wva = jnp.zeros((128, 1), jnp.float32).at[:_HID].set(wc)
    wvb = jnp.zeros((128, 1), jnp.float32).at[_HID:].set(wc)
    bv = bh[:, _ACT:_ACT + 1]

    logits3, value3 = _forward(
        x3, w1a, w1b, b1p, w2p, b2p, wla, wlb, bl, wva, wvb, bv, half=half)
    logits = logits3.reshape(2 * half, _ACT)[:B]
    value = value3.reshape(2 * half, 1)[:B]
    return logits, value


# speedup vs baseline: 2.7057x; 1.3414x over previous
import functools

import jax
import jax.numpy as jnp
from jax.experimental import pallas as pl
from jax.experimental.pallas import tpu as pltpu

_OBS = 8
_ACT = 4
_HID = 64
_TILE = 4096

_DN = (((0,), (0,)), ((), ()))


def _ac_kernel(x_ref, w1_ref, b1t_ref, w2_ref, b2t_ref,
               wa_ref, bat_ref, wc_ref, bct_ref,
               c0_ref, c1_ref, c2_ref, c3_ref, v_ref):
    x = x_ref[...]
    z1 = jax.lax.dot_general(
        w1_ref[...], x, (((0,), (1,)), ((), ())),
        preferred_element_type=jnp.float32)
    h1 = jnp.tanh(z1 + b1t_ref[...])
    z2 = jax.lax.dot_general(
        w2_ref[...], h1, _DN, preferred_element_type=jnp.float32)
    h2 = jnp.tanh(z2 + b2t_ref[...])
    lt = jax.lax.dot_general(
        wa_ref[...], h2, _DN, preferred_element_type=jnp.float32)
    lt = lt + bat_ref[...]
    vt = jax.lax.dot_general(
        wc_ref[...], h2, _DN, preferred_element_type=jnp.float32)
    v_ref[...] = vt + bct_ref[...]
    c0_ref[...] = lt[0:1]
    c1_ref[...] = lt[1:2]
    c2_ref[...] = lt[2:3]
    c3_ref[...] = lt[3:4]


@functools.partial(jax.jit, static_argnames=("bp",))
def _forward(x, w1c, b1t, w2c, b2t, wa, bat, wc, bct, *, bp):
    grid = (bp // _TILE,)
    row_spec = pl.BlockSpec((1, _TILE), lambda i: (0, i))
    outs = pl.pallas_call(
        _ac_kernel,
        grid=grid,
        in_specs=[
            pl.BlockSpec((_TILE, _OBS), lambda i: (i, 0)),
            pl.BlockSpec((_OBS, _HID), lambda i: (0, 0)),
            pl.BlockSpec((_HID, 1), lambda i: (0, 0)),
            pl.BlockSpec((_HID, _HID), lambda i: (0, 0)),
            pl.BlockSpec((_HID, 1), lambda i: (0, 0)),
            pl.BlockSpec((_HID, _ACT), lambda i: (0, 0)),
            pl.BlockSpec((_ACT, 1), lambda i: (0, 0)),
            pl.BlockSpec((_HID, 1), lambda i: (0, 0)),
            pl.BlockSpec((1, 1), lambda i: (0, 0)),
        ],
        out_specs=[row_spec, row_spec, row_spec, row_spec, row_spec],
        out_shape=[jax.ShapeDtypeStruct((1, bp), jnp.float32)
                   for _ in range(5)],
        compiler_params=pltpu.CompilerParams(
            dimension_semantics=("parallel",),
        ),
    )(x, w1c, b1t, w2c, b2t, wa, bat, wc, bct)
    return outs


def kernel(x, w1, b1, w2, b2, wh, bh):
    B = x.shape[0]
    bp = -(-B // _TILE) * _TILE
    if bp != B:
        x = jnp.pad(x, ((0, bp - B), (0, 0)))

    w1c = w1[:, :_HID]
    b1t = b1[:, :_HID].T
    w2c = w2[:_HID, :_HID]
    b2t = b2[:, :_HID].T
    wa = wh[:_HID, :_ACT]
    bat = bh[:, :_ACT].T
    wc = wh[:_HID, _ACT:_ACT + 1]
    bct = bh[:, _ACT:_ACT + 1]

    c0, c1, c2, c3, v = _forward(
        x, w1c, b1t, w2c, b2t, wa, bat, wc, bct, bp=bp)
    logits = jnp.concatenate(
        [c.reshape(bp, 1) for c in (c0, c1, c2, c3)], axis=1)
    value = v.reshape(bp, 1)
    if bp != B:
        logits = logits[:B]
        value = value[:B]
    return logits, value


# speedup vs baseline: 6.5517x; 2.4214x over previous
import functools

import jax
import jax.numpy as jnp
from jax.experimental import pallas as pl
from jax.experimental.pallas import tpu as pltpu

_OBS = 8
_ACT = 4
_HID = 64
_TILE = 4096

_DN = (((0,), (0,)), ((), ()))


def _ac_kernel(x_ref, w1_ref, b1t_ref, w2_ref, b2t_ref,
               wa_ref, bat_ref, wc_ref, bct_ref,
               c0_ref, c1_ref, c2_ref, c3_ref, v_ref):
    xt = x_ref[...]
    z1 = jax.lax.dot_general(
        w1_ref[...], xt, _DN,
        preferred_element_type=jnp.float32)
    h1 = jnp.tanh(z1 + b1t_ref[...])
    z2 = jax.lax.dot_general(
        w2_ref[...], h1, _DN, preferred_element_type=jnp.float32)
    h2 = jnp.tanh(z2 + b2t_ref[...])
    lt = jax.lax.dot_general(
        wa_ref[...], h2, _DN, preferred_element_type=jnp.float32)
    lt = lt + bat_ref[...]
    vt = jax.lax.dot_general(
        wc_ref[...], h2, _DN, preferred_element_type=jnp.float32)
    v_ref[...] = vt + bct_ref[...]
    c0_ref[...] = lt[0:1]
    c1_ref[...] = lt[1:2]
    c2_ref[...] = lt[2:3]
    c3_ref[...] = lt[3:4]


@functools.partial(jax.jit, static_argnames=("bp",))
def _forward(x, w1c, b1t, w2c, b2t, wa, bat, wc, bct, *, bp):
    grid = (bp // _TILE,)
    row_spec = pl.BlockSpec((1, _TILE), lambda i: (0, i))
    outs = pl.pallas_call(
        _ac_kernel,
        grid=grid,
        in_specs=[
            pl.BlockSpec((_OBS, _TILE), lambda i: (0, i)),
            pl.BlockSpec((_OBS, _HID), lambda i: (0, 0)),
            pl.BlockSpec((_HID, 1), lambda i: (0, 0)),
            pl.BlockSpec((_HID, _HID), lambda i: (0, 0)),
            pl.BlockSpec((_HID, 1), lambda i: (0, 0)),
            pl.BlockSpec((_HID, _ACT), lambda i: (0, 0)),
            pl.BlockSpec((_ACT, 1), lambda i: (0, 0)),
            pl.BlockSpec((_HID, 1), lambda i: (0, 0)),
            pl.BlockSpec((1, 1), lambda i: (0, 0)),
        ],
        out_specs=[row_spec, row_spec, row_spec, row_spec, row_spec],
        out_shape=[jax.ShapeDtypeStruct((1, bp), jnp.float32)
                   for _ in range(5)],
        compiler_params=pltpu.CompilerParams(
            dimension_semantics=("parallel",),
        ),
    )(x, w1c, b1t, w2c, b2t, wa, bat, wc, bct)
    return outs


def kernel(x, w1, b1, w2, b2, wh, bh):
    B = x.shape[0]
    bp = -(-B // _TILE) * _TILE
    if bp != B:
        x = jnp.pad(x, ((0, bp - B), (0, 0)))
    xt = x.T

    w1c = w1[:, :_HID]
    b1t = b1[:, :_HID].T
    w2c = w2[:_HID, :_HID]
    b2t = b2[:, :_HID].T
    wa = wh[:_HID, :_ACT]
    bat = bh[:, :_ACT].T
    wc = wh[:_HID, _ACT:_ACT + 1]
    bct = bh[:, _ACT:_ACT + 1]

    c0, c1, c2, c3, v = _forward(
        xt, w1c, b1t, w2c, b2t, wa, bat, wc, bct, bp=bp)
    logits = jnp.concatenate(
        [c.reshape(bp, 1) for c in (c0, c1, c2, c3)], axis=1)
    value = v.reshape(bp, 1)
    if bp != B:
        logits = logits[:B]
        value = value[:B]
    return logits, value


# speedup vs baseline: 9.1708x; 1.3998x over previous
import functools

import jax
import jax.numpy as jnp
from jax.experimental import pallas as pl
from jax.experimental.pallas import tpu as pltpu

_OBS = 8
_ACT = 4
_HID = 64
_TILE = 8192

_DN = (((0,), (0,)), ((), ()))


def _ac_kernel(x_ref, w1_ref, b1t_ref, w2_ref, b2t_ref,
               wa_ref, bat_ref, wc_ref, bct_ref,
               c0_ref, c1_ref, c2_ref, c3_ref, v_ref):
    xt = x_ref[...]
    z1 = jax.lax.dot_general(
        w1_ref[...], xt, _DN,
        preferred_element_type=jnp.float32)
    h1 = jnp.tanh(z1 + b1t_ref[...])
    z2 = jax.lax.dot_general(
        w2_ref[...], h1, _DN, preferred_element_type=jnp.float32)
    h2 = jnp.tanh(z2 + b2t_ref[...])
    lt = jax.lax.dot_general(
        wa_ref[...], h2, _DN, preferred_element_type=jnp.float32)
    lt = lt + bat_ref[...]
    vt = jax.lax.dot_general(
        wc_ref[...], h2, _DN, preferred_element_type=jnp.float32)
    v_ref[...] = vt + bct_ref[...]
    c0_ref[...] = lt[0:1]
    c1_ref[...] = lt[1:2]
    c2_ref[...] = lt[2:3]
    c3_ref[...] = lt[3:4]


@functools.partial(jax.jit, static_argnames=("bp",))
def _forward(x, w1c, b1t, w2c, b2t, wa, bat, wc, bct, *, bp):
    grid = (bp // _TILE,)
    row_spec = pl.BlockSpec((1, _TILE), lambda i: (0, i))
    outs = pl.pallas_call(
        _ac_kernel,
        grid=grid,
        in_specs=[
            pl.BlockSpec((_OBS, _TILE), lambda i: (0, i)),
            pl.BlockSpec((_OBS, _HID), lambda i: (0, 0)),
            pl.BlockSpec((_HID, 1), lambda i: (0, 0)),
            pl.BlockSpec((_HID, _HID), lambda i: (0, 0)),
            pl.BlockSpec((_HID, 1), lambda i: (0, 0)),
            pl.BlockSpec((_HID, _ACT), lambda i: (0, 0)),
            pl.BlockSpec((_ACT, 1), lambda i: (0, 0)),
            pl.BlockSpec((_HID, 1), lambda i: (0, 0)),
            pl.BlockSpec((1, 1), lambda i: (0, 0)),
        ],
        out_specs=[row_spec, row_spec, row_spec, row_spec, row_spec],
        out_shape=[jax.ShapeDtypeStruct((1, bp), jnp.float32)
                   for _ in range(5)],
        compiler_params=pltpu.CompilerParams(
            dimension_semantics=("parallel",),
        ),
    )(x, w1c, b1t, w2c, b2t, wa, bat, wc, bct)
    return outs


def kernel(x, w1, b1, w2, b2, wh, bh):
    B = x.shape[0]
    bp = -(-B // _TILE) * _TILE
    if bp != B:
        x = jnp.pad(x, ((0, bp - B), (0, 0)))
    xt = x.T

    w1c = w1[:, :_HID]
    b1t = b1[:, :_HID].T
    w2c = w2[:_HID, :_HID]
    b2t = b2[:, :_HID].T
    wa = wh[:_HID, :_ACT]
    bat = bh[:, :_ACT].T
    wc = wh[:_HID, _ACT:_ACT + 1]
    bct = bh[:, _ACT:_ACT + 1]

    c0, c1, c2, c3, v = _forward(
        xt, w1c, b1t, w2c, b2t, wa, bat, wc, bct, bp=bp)
    logits = jnp.concatenate(
        [c.reshape(bp, 1) for c in (c0, c1, c2, c3)], axis=1)
    value = v.reshape(bp, 1)
    if bp != B:
        logits = logits[:B]
        value = value[:B]
    return logits, value


# speedup vs baseline: 10.0737x; 1.0985x over previous
import functools

import jax
import jax.numpy as jnp
from jax.experimental import pallas as pl
from jax.experimental.pallas import tpu as pltpu

_OBS = 8
_ACT = 4
_HID = 64
_TILE = 16384

_DN = (((0,), (0,)), ((), ()))


def _ac_kernel(x_ref, w1_ref, b1t_ref, w2_ref, b2t_ref,
               wa_ref, bat_ref, wc_ref, bct_ref,
               c0_ref, c1_ref, c2_ref, c3_ref, v_ref):
    xt = x_ref[...]
    z1 = jax.lax.dot_general(
        w1_ref[...], xt, _DN,
        preferred_element_type=jnp.float32)
    h1 = jnp.tanh(z1 + b1t_ref[...])
    z2 = jax.lax.dot_general(
        w2_ref[...], h1, _DN, preferred_element_type=jnp.float32)
    h2 = jnp.tanh(z2 + b2t_ref[...])
    lt = jax.lax.dot_general(
        wa_ref[...], h2, _DN, preferred_element_type=jnp.float32)
    lt = lt + bat_ref[...]
    vt = jax.lax.dot_general(
        wc_ref[...], h2, _DN, preferred_element_type=jnp.float32)
    v_ref[...] = vt + bct_ref[...]
    c0_ref[...] = lt[0:1]
    c1_ref[...] = lt[1:2]
    c2_ref[...] = lt[2:3]
    c3_ref[...] = lt[3:4]


@functools.partial(jax.jit, static_argnames=("bp",))
def _forward(x, w1c, b1t, w2c, b2t, wa, bat, wc, bct, *, bp):
    grid = (bp // _TILE,)
    row_spec = pl.BlockSpec((1, _TILE), lambda i: (0, i))
    outs = pl.pallas_call(
        _ac_kernel,
        grid=grid,
        in_specs=[
            pl.BlockSpec((_OBS, _TILE), lambda i: (0, i)),
            pl.BlockSpec((_OBS, _HID), lambda i: (0, 0)),
            pl.BlockSpec((_HID, 1), lambda i: (0, 0)),
            pl.BlockSpec((_HID, _HID), lambda i: (0, 0)),
            pl.BlockSpec((_HID, 1), lambda i: (0, 0)),
            pl.BlockSpec((_HID, _ACT), lambda i: (0, 0)),
            pl.BlockSpec((_ACT, 1), lambda i: (0, 0)),
            pl.BlockSpec((_HID, 1), lambda i: (0, 0)),
            pl.BlockSpec((1, 1), lambda i: (0, 0)),
        ],
        out_specs=[row_spec, row_spec, row_spec, row_spec, row_spec],
        out_shape=[jax.ShapeDtypeStruct((1, bp), jnp.float32)
                   for _ in range(5)],
        compiler_params=pltpu.CompilerParams(
            dimension_semantics=("parallel",),
        ),
    )(x, w1c, b1t, w2c, b2t, wa, bat, wc, bct)
    return outs


def kernel(x, w1, b1, w2, b2, wh, bh):
    B = x.shape[0]
    bp = -(-B // _TILE) * _TILE
    if bp != B:
        x = jnp.pad(x, ((0, bp - B), (0, 0)))
    xt = x.T

    w1c = w1[:, :_HID]
    b1t = b1[:, :_HID].T
    w2c = w2[:_HID, :_HID]
    b2t = b2[:, :_HID].T
    wa = wh[:_HID, :_ACT]
    bat = bh[:, :_ACT].T
    wc = wh[:_HID, _ACT:_ACT + 1]
    bct = bh[:, _ACT:_ACT + 1]

    c0, c1, c2, c3, v = _forward(
        xt, w1c, b1t, w2c, b2t, wa, bat, wc, bct, bp=bp)
    logits = jnp.concatenate(
        [c.reshape(bp, 1) for c in (c0, c1, c2, c3)], axis=1)
    value = v.reshape(bp, 1)
    if bp != B:
        logits = logits[:B]
        value = value[:B]
    return logits, value


# speedup vs baseline: 10.5313x; 1.0454x over previous
import functools

import jax
import jax.numpy as jnp
from jax.experimental import pallas as pl
from jax.experimental.pallas import tpu as pltpu

_OBS = 8
_ACT = 4
_HID = 64
_TILE = 32768

_DN = (((0,), (0,)), ((), ()))


def _ac_kernel(x_ref, w1_ref, b1t_ref, w2_ref, b2t_ref,
               wa_ref, bat_ref, wc_ref, bct_ref,
               c0_ref, c1_ref, c2_ref, c3_ref, v_ref):
    xt = x_ref[...]
    z1 = jax.lax.dot_general(
        w1_ref[...], xt, _DN,
        preferred_element_type=jnp.float32)
    h1 = jnp.tanh(z1 + b1t_ref[...])
    z2 = jax.lax.dot_general(
        w2_ref[...], h1, _DN, preferred_element_type=jnp.float32)
    h2 = jnp.tanh(z2 + b2t_ref[...])
    lt = jax.lax.dot_general(
        wa_ref[...], h2, _DN, preferred_element_type=jnp.float32)
    lt = lt + bat_ref[...]
    vt = jax.lax.dot_general(
        wc_ref[...], h2, _DN, preferred_element_type=jnp.float32)
    v_ref[...] = vt + bct_ref[...]
    c0_ref[...] = lt[0:1]
    c1_ref[...] = lt[1:2]
    c2_ref[...] = lt[2:3]
    c3_ref[...] = lt[3:4]


@functools.partial(jax.jit, static_argnames=("bp",))
def _forward(x, w1c, b1t, w2c, b2t, wa, bat, wc, bct, *, bp):
    grid = (bp // _TILE,)
    row_spec = pl.BlockSpec((1, _TILE), lambda i: (0, i))
    outs = pl.pallas_call(
        _ac_kernel,
        grid=grid,
        in_specs=[
            pl.BlockSpec((_OBS, _TILE), lambda i: (0, i)),
            pl.BlockSpec((_OBS, _HID), lambda i: (0, 0)),
            pl.BlockSpec((_HID, 1), lambda i: (0, 0)),
            pl.BlockSpec((_HID, _HID), lambda i: (0, 0)),
            pl.BlockSpec((_HID, 1), lambda i: (0, 0)),
            pl.BlockSpec((_HID, _ACT), lambda i: (0, 0)),
            pl.BlockSpec((_ACT, 1), lambda i: (0, 0)),
            pl.BlockSpec((_HID, 1), lambda i: (0, 0)),
            pl.BlockSpec((1, 1), lambda i: (0, 0)),
        ],
        out_specs=[row_spec, row_spec, row_spec, row_spec, row_spec],
        out_shape=[jax.ShapeDtypeStruct((1, bp), jnp.float32)
                   for _ in range(5)],
        compiler_params=pltpu.CompilerParams(
            dimension_semantics=("parallel",),
        ),
    )(x, w1c, b1t, w2c, b2t, wa, bat, wc, bct)
    return outs


def kernel(x, w1, b1, w2, b2, wh, bh):
    B = x.shape[0]
    bp = -(-B // _TILE) * _TILE
    if bp != B:
        x = jnp.pad(x, ((0, bp - B), (0, 0)))
    xt = x.T

    w1c = w1[:, :_HID]
    b1t = b1[:, :_HID].T
    w2c = w2[:_HID, :_HID]
    b2t = b2[:, :_HID].T
    wa = wh[:_HID, :_ACT]
    bat = bh[:, :_ACT].T
    wc = wh[:_HID, _ACT:_ACT + 1]
    bct = bh[:, _ACT:_ACT + 1]

    c0, c1, c2, c3, v = _forward(
        xt, w1c, b1t, w2c, b2t, wa, bat, wc, bct, bp=bp)
    logits = jnp.concatenate(
        [c.reshape(bp, 1) for c in (c0, c1, c2, c3)], axis=1)
    value = v.reshape(bp, 1)
    if bp != B:
        logits = logits[:B]
        value = value[:B]
    return logits, value
